# Initial kernel scaffold; baseline (speedup 1.0000x reference)
import jax
import jax.numpy as jnp
from jax.experimental import pallas as pl


def kernel(H, A, loop_w, W1, Wp, p, Wu, W2):
    raise NotImplementedError("write your pallas kernel here")


# speedup vs baseline: 1.0571x; 1.0571x over previous
import functools

import jax
import jax.numpy as jnp
from jax.experimental import pallas as pl

N = 4096
BR = 512
NB = N // BR
K = 2048


def _deg_body(a_ref, deg_ref):
    deg_ref[...] = jnp.sum(a_ref[...], axis=1, keepdims=True)


def _gcn_body(deg_ref, lw_ref, hin_ref, w_ref, a_ref, out_ref,
              x_scr, z_scr, dinv_scr, *, last):
    i = pl.program_id(0)

    @pl.when(i == 0)
    def _prologue():
        dg = deg_ref[...]
        dinv = jnp.where(dg > 0.0, jax.lax.rsqrt(dg), 0.0)
        dinv_scr[...] = dinv
        x = jnp.dot(hin_ref[...], w_ref[...],
                    preferred_element_type=jnp.float32)
        x_scr[...] = x
        z_scr[...] = x * dinv

    acc = jnp.dot(a_ref[...], z_scr[...], preferred_element_type=jnp.float32)
    dv = dinv_scr[pl.ds(i * BR, BR), :]
    xb = x_scr[pl.ds(i * BR, BR), :]
    h = jnp.maximum(dv * acc + lw_ref[...] * xb, 0.0)
    if last:
        m = jnp.max(h, axis=1, keepdims=True)
        e = jnp.exp(h - m)
        lse = jnp.log(jnp.sum(e, axis=1, keepdims=True)) + m
        out_ref[...] = h - lse
    else:
        out_ref[...] = h


def _gcn_scores_body(deg_ref, lw_ref, hin_ref, w_ref, p_ref, a_ref,
                     out_ref, s_ref, x_scr, z_scr, dinv_scr):
    _gcn_body(deg_ref, lw_ref, hin_ref, w_ref, a_ref, out_ref,
              x_scr, z_scr, dinv_scr, last=False)
    h = out_ref[...]
    pvec = p_ref[...]
    pn = jnp.sqrt(jnp.sum(pvec * pvec)) + 1e-12
    s_ref[...] = jnp.dot(h, pvec, preferred_element_type=jnp.float32) / pn


def _gcn_skip_body(deg_ref, lw_ref, h1_ref, hp_ref, gate_ref, w_ref, a_ref,
                   out_ref, x_scr, z_scr, dinv_scr):
    i = pl.program_id(0)

    @pl.when(i == 0)
    def _prologue():
        dg = deg_ref[...]
        dinv = jnp.where(dg > 0.0, jax.lax.rsqrt(dg), 0.0)
        dinv_scr[...] = dinv
        h2 = h1_ref[...] + gate_ref[...] * hp_ref[...]
        x = jnp.dot(h2, w_ref[...], preferred_element_type=jnp.float32)
        x_scr[...] = x
        z_scr[...] = x * dinv

    acc = jnp.dot(a_ref[...], z_scr[...], preferred_element_type=jnp.float32)
    dv = dinv_scr[pl.ds(i * BR, BR), :]
    xb = x_scr[pl.ds(i * BR, BR), :]
    out_ref[...] = jnp.maximum(dv * acc + lw_ref[...] * xb, 0.0)


def _mask_body(s_ref, gate_ref):
    s = s_ref[...] + 0.0
    b = jax.lax.bitcast_convert_type(s, jnp.int32)
    imin = jnp.int32(-2147483648)
    key = jnp.where(b >= 0, b, imin - b)

    def tstep(j, t):
        q = t + (jnp.int32(1) << (jnp.int32(30) - j))
        cnt = jnp.sum(jnp.where(key >= q, 1, 0).astype(jnp.int32))
        return jnp.where(cnt >= K, q, t)

    t = jax.lax.fori_loop(0, 31, tstep, imin)

    greater = key > t
    eq = key == t
    rem = K - jnp.sum(jnp.where(greater, 1, 0).astype(jnp.int32))
    ri = jax.lax.broadcasted_iota(jnp.int32, s.shape, 0)
    ci = jax.lax.broadcasted_iota(jnp.int32, s.shape, 1)
    idx = ri * s.shape[1] + ci

    def mstep(j, m):
        q = m + (jnp.int32(1) << (jnp.int32(12) - j))
        cnt = jnp.sum(jnp.where(eq & (idx < q), 1, 0).astype(jnp.int32))
        return jnp.where(cnt <= rem, q, m)

    mm = jax.lax.fori_loop(0, 13, mstep, jnp.int32(0))

    mask = greater | (eq & (idx < mm))
    gate_ref[...] = jnp.where(mask, jax.nn.sigmoid(s_ref[...]),
                              jnp.float32(0.0))


def _a_spec():
    return pl.BlockSpec((BR, N), lambda i: (i, 0))


def _full(shape):
    return pl.BlockSpec(shape, lambda i: (0, 0))


def _row_spec(d):
    return pl.BlockSpec((BR, d), lambda i: (i, 0))


def _gcn_scratch(dout):
    return [
        pltpu_vmem((N, dout)),
        pltpu_vmem((N, dout)),
        pltpu_vmem((N, 1)),
    ]


def pltpu_vmem(shape):
    from jax.experimental.pallas import tpu as pltpu
    return pltpu.VMEM(shape, jnp.float32)


def _gcn_pass(A, deg, lw, Hin, W, *, last=False):
    din, dout = W.shape
    body = functools.partial(_gcn_body, last=last)
    return pl.pallas_call(
        body,
        grid=(NB,),
        in_specs=[_full((N, 1)), _row_spec(1), _full((N, din)),
                  _full((din, dout)), _a_spec()],
        out_specs=_row_spec(dout),
        out_shape=jax.ShapeDtypeStruct((N, dout), jnp.float32),
        scratch_shapes=_gcn_scratch(dout),
    )(deg, lw, Hin, W, A)


def _gcn_scores_pass(A, deg, lw, Hin, W, p2):
    din, dout = W.shape
    return pl.pallas_call(
        _gcn_scores_body,
        grid=(NB,),
        in_specs=[_full((N, 1)), _row_spec(1), _full((N, din)),
                  _full((din, dout)), _full((dout, 1)), _a_spec()],
        out_specs=(_row_spec(dout), _row_spec(1)),
        out_shape=(jax.ShapeDtypeStruct((N, dout), jnp.float32),
                   jax.ShapeDtypeStruct((N, 1), jnp.float32)),
        scratch_shapes=_gcn_scratch(dout),
    )(deg, lw, Hin, W, p2, A)


def _gcn_skip_pass(A, deg, lw, H1, Hp, gate, W):
    din, dout = W.shape
    return pl.pallas_call(
        _gcn_skip_body,
        grid=(NB,),
        in_specs=[_full((N, 1)), _row_spec(1), _full((N, din)),
                  _full((N, din)), _full((N, 1)), _full((din, dout)),
                  _a_spec()],
        out_specs=_row_spec(dout),
        out_shape=jax.ShapeDtypeStruct((N, dout), jnp.float32),
        scratch_shapes=_gcn_scratch(dout),
    )(deg, lw, H1, Hp, gate, W, A)


def kernel(H, A, loop_w, W1, Wp, p, Wu, W2):
    lw = loop_w.reshape(N, 1)
    p2 = p.reshape(-1, 1)

    deg = pl.pallas_call(
        _deg_body,
        grid=(NB,),
        in_specs=[_a_spec()],
        out_specs=_row_spec(1),
        out_shape=jax.ShapeDtypeStruct((N, 1), jnp.float32),
    )(A)

    H1 = _gcn_pass(A, deg, lw, H, W1)
    Hp, scores = _gcn_scores_pass(A, deg, lw, H1, Wp, p2)

    s32 = scores.reshape(32, 128)
    gate32 = pl.pallas_call(
        _mask_body,
        out_shape=jax.ShapeDtypeStruct((32, 128), jnp.float32),
    )(s32)
    gate = gate32.reshape(N, 1)

    H3 = _gcn_skip_pass(A, deg, lw, H1, Hp, gate, Wu)
    out = _gcn_pass(A, deg, lw, H3, W2, last=True)
    return out


# speedup vs baseline: 1.1783x; 1.1147x over previous
import functools

import jax
import jax.numpy as jnp
from jax.experimental import pallas as pl

N = 4096
BR = 512
NB = N // BR
K = 2048


def _deg_body(a_ref, deg_ref, ab_ref):
    a = a_ref[...]
    deg_ref[...] = jnp.sum(a, axis=1, keepdims=True)
    ab_ref[...] = a.astype(jnp.bfloat16)


def _gcn_body(deg_ref, lw_ref, hin_ref, w_ref, a_ref, out_ref,
              x_scr, z_scr, dinv_scr, *, last):
    i = pl.program_id(0)

    @pl.when(i == 0)
    def _prologue():
        dg = deg_ref[...]
        dinv = jnp.where(dg > 0.0, jax.lax.rsqrt(dg), 0.0)
        dinv_scr[...] = dinv
        x = jnp.dot(hin_ref[...], w_ref[...],
                    preferred_element_type=jnp.float32)
        x_scr[...] = x
        z_scr[...] = (x * dinv).astype(jnp.bfloat16)

    acc = jnp.dot(a_ref[...], z_scr[...], preferred_element_type=jnp.float32)
    dv = dinv_scr[pl.ds(i * BR, BR), :]
    xb = x_scr[pl.ds(i * BR, BR), :]
    h = jnp.maximum(dv * acc + lw_ref[...] * xb, 0.0)
    if last:
        m = jnp.max(h, axis=1, keepdims=True)
        e = jnp.exp(h - m)
        lse = jnp.log(jnp.sum(e, axis=1, keepdims=True)) + m
        out_ref[...] = h - lse
    else:
        out_ref[...] = h


def _gcn_scores_body(deg_ref, lw_ref, hin_ref, w_ref, p_ref, a_ref,
                     out_ref, s_ref, x_scr, z_scr, dinv_scr):
    _gcn_body(deg_ref, lw_ref, hin_ref, w_ref, a_ref, out_ref,
              x_scr, z_scr, dinv_scr, last=False)
    h = out_ref[...]
    pvec = p_ref[...]
    pn = jnp.sqrt(jnp.sum(pvec * pvec)) + 1e-12
    s_ref[...] = jnp.dot(h, pvec, preferred_element_type=jnp.float32) / pn


def _gcn_skip_body(deg_ref, lw_ref, h1_ref, hp_ref, gate_ref, w_ref, a_ref,
                   out_ref, x_scr, z_scr, dinv_scr):
    i = pl.program_id(0)

    @pl.when(i == 0)
    def _prologue():
        dg = deg_ref[...]
        dinv = jnp.where(dg > 0.0, jax.lax.rsqrt(dg), 0.0)
        dinv_scr[...] = dinv
        h2 = h1_ref[...] + gate_ref[...] * hp_ref[...]
        x = jnp.dot(h2, w_ref[...], preferred_element_type=jnp.float32)
        x_scr[...] = x
        z_scr[...] = (x * dinv).astype(jnp.bfloat16)

    acc = jnp.dot(a_ref[...], z_scr[...], preferred_element_type=jnp.float32)
    dv = dinv_scr[pl.ds(i * BR, BR), :]
    xb = x_scr[pl.ds(i * BR, BR), :]
    out_ref[...] = jnp.maximum(dv * acc + lw_ref[...] * xb, 0.0)


def _mask_body(s_ref, gate_ref):
    s = s_ref[...] + 0.0
    b = jax.lax.bitcast_convert_type(s, jnp.int32)
    imin = jnp.int32(-2147483648)
    key = jnp.where(b >= 0, b, imin - b)

    def tstep(j, t):
        q = t + (jnp.int32(1) << (jnp.int32(30) - j))
        cnt = jnp.sum(jnp.where(key >= q, 1, 0).astype(jnp.int32))
        return jnp.where(cnt >= K, q, t)

    t = jax.lax.fori_loop(0, 31, tstep, imin)

    greater = key > t
    eq = key == t
    rem = K - jnp.sum(jnp.where(greater, 1, 0).astype(jnp.int32))
    ri = jax.lax.broadcasted_iota(jnp.int32, s.shape, 0)
    ci = jax.lax.broadcasted_iota(jnp.int32, s.shape, 1)
    idx = ri * s.shape[1] + ci

    def mstep(j, m):
        q = m + (jnp.int32(1) << (jnp.int32(12) - j))
        cnt = jnp.sum(jnp.where(eq & (idx < q), 1, 0).astype(jnp.int32))
        return jnp.where(cnt <= rem, q, m)

    mm = jax.lax.fori_loop(0, 13, mstep, jnp.int32(0))

    mask = greater | (eq & (idx < mm))
    gate_ref[...] = jnp.where(mask, jax.nn.sigmoid(s_ref[...]),
                              jnp.float32(0.0))


def _a_spec():
    return pl.BlockSpec((BR, N), lambda i: (i, 0))


def _full(shape):
    return pl.BlockSpec(shape, lambda i: (0, 0))


def _row_spec(d):
    return pl.BlockSpec((BR, d), lambda i: (i, 0))


def _gcn_scratch(dout):
    return [
        pltpu_vmem((N, dout), jnp.float32),
        pltpu_vmem((N, dout), jnp.bfloat16),
        pltpu_vmem((N, 1), jnp.float32),
    ]


def pltpu_vmem(shape, dtype):
    from jax.experimental.pallas import tpu as pltpu
    return pltpu.VMEM(shape, dtype)


def _gcn_pass(A, deg, lw, Hin, W, *, last=False):
    din, dout = W.shape
    body = functools.partial(_gcn_body, last=last)
    return pl.pallas_call(
        body,
        grid=(NB,),
        in_specs=[_full((N, 1)), _row_spec(1), _full((N, din)),
                  _full((din, dout)), _a_spec()],
        out_specs=_row_spec(dout),
        out_shape=jax.ShapeDtypeStruct((N, dout), jnp.float32),
        scratch_shapes=_gcn_scratch(dout),
    )(deg, lw, Hin, W, A)


def _gcn_scores_pass(A, deg, lw, Hin, W, p2):
    din, dout = W.shape
    return pl.pallas_call(
        _gcn_scores_body,
        grid=(NB,),
        in_specs=[_full((N, 1)), _row_spec(1), _full((N, din)),
                  _full((din, dout)), _full((dout, 1)), _a_spec()],
        out_specs=(_row_spec(dout), _row_spec(1)),
        out_shape=(jax.ShapeDtypeStruct((N, dout), jnp.float32),
                   jax.ShapeDtypeStruct((N, 1), jnp.float32)),
        scratch_shapes=_gcn_scratch(dout),
    )(deg, lw, Hin, W, p2, A)


def _gcn_skip_pass(A, deg, lw, H1, Hp, gate, W):
    din, dout = W.shape
    return pl.pallas_call(
        _gcn_skip_body,
        grid=(NB,),
        in_specs=[_full((N, 1)), _row_spec(1), _full((N, din)),
                  _full((N, din)), _full((N, 1)), _full((din, dout)),
                  _a_spec()],
        out_specs=_row_spec(dout),
        out_shape=jax.ShapeDtypeStruct((N, dout), jnp.float32),
        scratch_shapes=_gcn_scratch(dout),
    )(deg, lw, H1, Hp, gate, W, A)


def kernel(H, A, loop_w, W1, Wp, p, Wu, W2):
    lw = loop_w.reshape(N, 1)
    p2 = p.reshape(-1, 1)

    deg, Ab = pl.pallas_call(
        _deg_body,
        grid=(NB,),
        in_specs=[_a_spec()],
        out_specs=(_row_spec(1), pl.BlockSpec((BR, N), lambda i: (i, 0))),
        out_shape=(jax.ShapeDtypeStruct((N, 1), jnp.float32),
                   jax.ShapeDtypeStruct((N, N), jnp.bfloat16)),
    )(A)

    H1 = _gcn_pass(Ab, deg, lw, H, W1)
    Hp, scores = _gcn_scores_pass(Ab, deg, lw, H1, Wp, p2)

    s32 = scores.reshape(32, 128)
    gate32 = pl.pallas_call(
        _mask_body,
        out_shape=jax.ShapeDtypeStruct((32, 128), jnp.float32),
    )(s32)
    gate = gate32.reshape(N, 1)

    H3 = _gcn_skip_pass(Ab, deg, lw, H1, Hp, gate, Wu)
    out = _gcn_pass(Ab, deg, lw, H3, W2, last=True)
    return out
